# baseline (speedup 1.0000x reference)
import functools

import jax
import jax.numpy as jnp
from jax import lax
from jax.experimental import pallas as pl
from jax.experimental.pallas import tpu as pltpu
from jax.experimental.pallas import tpu_sc as plsc

C, G, D = 32, 2048, 256
GENE_V = 60000
_NC, _NS = 2, 16
_NW = _NC * _NS
_CH = 128


def _silu(x):
    return x * jax.nn.sigmoid(x)


def _sc_gather(gene_table_i32, gene_idx, cond_table, cond_idx):
    B = gene_idx.shape[0]
    CB = cond_idx.shape[0]
    H = gene_table_i32.shape[1]
    rows_w = B // _NW
    nch = rows_w // _CH

    mesh = plsc.VectorSubcoreMesh(core_axis_name="c", subcore_axis_name="s")

    @functools.partial(
        pl.kernel,
        out_type=(
            jax.ShapeDtypeStruct((B, H), jnp.int32),
            jax.ShapeDtypeStruct((CB, D), jnp.float32),
        ),
        mesh=mesh,
        scratch_types=[
            pltpu.VMEM((rows_w,), jnp.int32),
            pltpu.VMEM((_CH, H), jnp.int32),
            pltpu.VMEM((_CH, H), jnp.int32),
            pltpu.VMEM((CB,), jnp.int32),
            pltpu.VMEM((CB, D), jnp.float32),
            pltpu.SemaphoreType.DMA,
            pltpu.SemaphoreType.DMA,
            pltpu.SemaphoreType.DMA,
            pltpu.SemaphoreType.DMA,
            pltpu.SemaphoreType.DMA,
        ],
    )
    def gather_k(table_h, idx_h, ctab_h, cidx_h, out_h, cout_h,
                 idx_v, rows_a, rows_b, cidx_v, crows_v,
                 gsem_a, gsem_b, ssem_a, ssem_b, csem):
        wid = lax.axis_index("s") * _NC + lax.axis_index("c")
        base = wid * rows_w
        pltpu.sync_copy(idx_h.at[pl.ds(base, rows_w)], idx_v)

        bufs = (rows_a, rows_b)
        gsems = (gsem_a, gsem_b)
        ssems = (ssem_a, ssem_b)
        gcp = [None, None]
        scp = [None, None]
        for k in range(2):
            gcp[k] = pltpu.async_copy(
                table_h.at[idx_v.at[pl.ds(k * _CH, _CH)]], bufs[k], gsems[k])
        for k in range(nch):
            b = k % 2
            gcp[b].wait()
            scp[b] = pltpu.async_copy(
                bufs[b], out_h.at[pl.ds(base + k * _CH, _CH)], ssems[b])
            if k + 2 < nch:
                scp[b].wait()
                gcp[b] = pltpu.async_copy(
                    table_h.at[idx_v.at[pl.ds((k + 2) * _CH, _CH)]],
                    bufs[b], gsems[b])
        scp[0].wait()
        scp[1].wait()

        @pl.when(wid == 0)
        def _():
            pltpu.sync_copy(cidx_h, cidx_v)
            pltpu.async_copy(ctab_h.at[cidx_v], crows_v, csem).wait()
            pltpu.sync_copy(crows_v, cout_h)

    return gather_k(gene_table_i32, gene_idx, cond_table, cond_idx)


def _cell_body(gene_ref, ev_ref, tf_ref, validt_ref, ce_ref,
               eW1_ref, eb1_ref, eW2_ref, eb2_ref, eW3_ref, eb3_ref,
               mW1_ref, mb1_ref, mW2_ref, mb2_ref,
               tf_tab_ref, gene_type_ref, zero_ref,
               ctxW1_ref, ctxb1_ref, ctxW2_ref, ctxb2_ref,
               pW1_ref, pb1_ref, pW2_ref, pb2_ref,
               xW1_ref, xb1_ref, xW2_ref, xb2_ref,
               prefix_type_ref, rms_ref, pos_ref, out_ref):
    f32 = jnp.float32
    bf16 = jnp.bfloat16
    v = ev_ref[0]
    h = _silu(v * eW1_ref[...] + eb1_ref[...])
    h = _silu(jnp.dot(h.astype(bf16), eW2_ref[...],
                      preferred_element_type=f32) + eb2_ref[...])
    expr = jnp.dot(h.astype(bf16), eW3_ref[...],
                   preferred_element_type=f32) + eb3_ref[...]
    expr = jnp.where(v == 0.0, zero_ref[...], expr)
    m = _silu(v * mW1_ref[...] + mb1_ref[...])
    mod = jnp.dot(m.astype(bf16), mW2_ref[...],
                  preferred_element_type=f32) + mb2_ref[...]
    scale = jax.nn.sigmoid(mod[:, :D])
    shift = mod[:, D:]
    t0 = tf_tab_ref[0:1, :]
    t1 = tf_tab_ref[1:2, :]
    tf_emb = t0 + tf_ref[0] * (t1 - t0)
    gene = gene_ref[0].astype(f32)
    tokens = (gene + expr + tf_emb + gene_type_ref[...]) * scale + shift
    vrow = validt_ref[0]
    pooled = jnp.dot(vrow, tokens, preferred_element_type=f32) / jnp.maximum(
        jnp.sum(vrow), 1.0)
    ctx = jnp.dot(_silu(jnp.dot(pooled, ctxW1_ref[...],
                                preferred_element_type=f32) + ctxb1_ref[...]),
                  ctxW2_ref[...], preferred_element_type=f32) + ctxb2_ref[...]
    ce = ce_ref[0]
    ptok = jnp.dot(_silu(jnp.dot(ce, pW1_ref[...],
                                 preferred_element_type=f32) + pb1_ref[...]),
                   pW2_ref[...], preferred_element_type=f32) + pb2_ref[...]
    cbias = jnp.dot(_silu(jnp.dot(ce, xW1_ref[...],
                                  preferred_element_type=f32) + xb1_ref[...]),
                    xW2_ref[...], preferred_element_type=f32) + xb2_ref[...]
    prefix_row = ptok + ctx + prefix_type_ref[...] + pos_ref[0:1, :]
    genes = tokens + cbias + pos_ref[1:, :]
    full = jnp.concatenate([prefix_row, genes], axis=0)
    norm = full * lax.rsqrt(
        jnp.mean(full * full, axis=-1, keepdims=True) + 1e-6) * rms_ref[...]
    out_ref[0] = norm


def kernel(expression_values, gene_table, zero_embedding, eW1, eb1, eW2, eb2,
           eW3, eb3, mW1, mb1, mW2, mb2, cond_table, pW1, pb1, pW2, pb2,
           xW1, xb1, xW2, xb2, ctxW1, ctxb1, ctxW2, ctxb2, tf_table,
           pos_table, prefix_type, gene_type, rms_w, input_ids,
           condition_ids, padding_mask, non_tf_mask):
    gene_idx = input_ids.reshape(-1).astype(jnp.int32)
    cond_idx = condition_ids.reshape(-1).astype(jnp.int32)
    gene_table_i32 = jax.lax.bitcast_convert_type(
        gene_table.astype(jnp.bfloat16).reshape(GENE_V, D // 2, 2),
        jnp.int32)
    gathered_i32, ce_rows = _sc_gather(gene_table_i32, gene_idx,
                                       cond_table, cond_idx)
    gathered = jax.lax.bitcast_convert_type(
        gathered_i32, jnp.bfloat16).reshape(C, G, D)
    ce3 = ce_rows.reshape(C, 1, 4 * D)

    ev3 = expression_values.reshape(C, G, 1)
    tf3 = non_tf_mask.astype(jnp.float32).reshape(C, G, 1)
    validt = (~padding_mask).astype(jnp.float32).reshape(C, 1, G)
    pos = pos_table[: G + 1]

    row = lambda b: b.reshape(1, -1)
    bf = lambda w: w.astype(jnp.bfloat16)
    weights = (eW1, row(eb1), bf(eW2), row(eb2), bf(eW3), row(eb3),
               mW1, row(mb1), bf(mW2), row(mb2),
               tf_table, gene_type.reshape(1, D), row(zero_embedding),
               ctxW1, row(ctxb1), ctxW2, row(ctxb2),
               pW1, row(pb1), pW2, row(pb2),
               xW1, row(xb1), xW2, row(xb2),
               prefix_type.reshape(1, D), row(rms_w), pos)

    full_spec = lambda a: pl.BlockSpec(a.shape, lambda c: (0,) * a.ndim)
    in_specs = [
        pl.BlockSpec((1, G, D), lambda c: (c, 0, 0)),
        pl.BlockSpec((1, G, 1), lambda c: (c, 0, 0)),
        pl.BlockSpec((1, G, 1), lambda c: (c, 0, 0)),
        pl.BlockSpec((1, 1, G), lambda c: (c, 0, 0)),
        pl.BlockSpec((1, 1, 4 * D), lambda c: (c, 0, 0)),
    ] + [full_spec(w) for w in weights]

    out = pl.pallas_call(
        _cell_body,
        grid=(C,),
        in_specs=in_specs,
        out_specs=pl.BlockSpec((1, G + 1, D), lambda c: (c, 0, 0)),
        out_shape=jax.ShapeDtypeStruct((C, G + 1, D), jnp.float32),
    )(gathered, ev3, tf3, validt, ce3, *weights)
    return out


# speedup vs baseline: 3.1717x; 3.1717x over previous
import functools

import jax
import jax.numpy as jnp
from jax import lax
from jax.experimental import pallas as pl
from jax.experimental.pallas import tpu as pltpu
from jax.experimental.pallas import tpu_sc as plsc

C, G, D = 32, 2048, 256
_NC, _NS = 2, 16
_NW = _NC * _NS
_CH = 128
_TB = 2048


def _silu(x):
    return x * jax.nn.sigmoid(x)


def _sc_gather(gene_table, gene_idx, cond_table, cond_idx):
    B = gene_idx.shape[0]
    CB = cond_idx.shape[0]
    rows_w = B // _NW
    nch = rows_w // _CH

    mesh = plsc.VectorSubcoreMesh(core_axis_name="c", subcore_axis_name="s")

    @functools.partial(
        pl.kernel,
        out_type=(
            jax.ShapeDtypeStruct((B, D), jnp.float32),
            jax.ShapeDtypeStruct((CB, D), jnp.float32),
        ),
        mesh=mesh,
        scratch_types=[
            pltpu.VMEM((rows_w,), jnp.int32),
            pltpu.VMEM((_CH, D), jnp.float32),
            pltpu.VMEM((_CH, D), jnp.float32),
            pltpu.VMEM((CB,), jnp.int32),
            pltpu.VMEM((CB, D), jnp.float32),
            pltpu.SemaphoreType.DMA,
            pltpu.SemaphoreType.DMA,
            pltpu.SemaphoreType.DMA,
            pltpu.SemaphoreType.DMA,
            pltpu.SemaphoreType.DMA,
        ],
    )
    def gather_k(table_h, idx_h, ctab_h, cidx_h, out_h, cout_h,
                 idx_v, rows_a, rows_b, cidx_v, crows_v,
                 gsem_a, gsem_b, ssem_a, ssem_b, csem):
        wid = lax.axis_index("s") * _NC + lax.axis_index("c")
        base = wid * rows_w
        pltpu.sync_copy(idx_h.at[pl.ds(base, rows_w)], idx_v)

        bufs = (rows_a, rows_b)
        gsems = (gsem_a, gsem_b)
        ssems = (ssem_a, ssem_b)
        gcp = [None, None]
        scp = [None, None]
        for k in range(2):
            gcp[k] = pltpu.async_copy(
                table_h.at[idx_v.at[pl.ds(k * _CH, _CH)]], bufs[k], gsems[k])
        for k in range(nch):
            b = k % 2
            gcp[b].wait()
            scp[b] = pltpu.async_copy(
                bufs[b], out_h.at[pl.ds(base + k * _CH, _CH)], ssems[b])
            if k + 2 < nch:
                scp[b].wait()
                gcp[b] = pltpu.async_copy(
                    table_h.at[idx_v.at[pl.ds((k + 2) * _CH, _CH)]],
                    bufs[b], gsems[b])
        scp[0].wait()
        scp[1].wait()

        @pl.when(wid == 0)
        def _():
            pltpu.sync_copy(cidx_h, cidx_v)
            pltpu.async_copy(ctab_h.at[cidx_v], crows_v, csem).wait()
            pltpu.sync_copy(crows_v, cout_h)

    return gather_k(gene_table, gene_idx, cond_table, cond_idx)


def _mlp_body(ev_ref, tf_ref,
              eW1_ref, eb1_ref, eW2_ref, eb2_ref, eW3_ref, eb3_ref,
              mW1_ref, mb1_ref, mW2_ref, mb2_ref,
              tf_tab_ref, gene_type_ref, zero_ref,
              scale_ref, partial_ref):
    f32 = jnp.float32
    bf16 = jnp.bfloat16
    v = ev_ref[...]
    h = _silu(v * eW1_ref[...] + eb1_ref[...])
    h = _silu(jnp.dot(h.astype(bf16), eW2_ref[...],
                      preferred_element_type=f32) + eb2_ref[...])
    expr = jnp.dot(h.astype(bf16), eW3_ref[...],
                   preferred_element_type=f32) + eb3_ref[...]
    expr = jnp.where(v == 0.0, zero_ref[...], expr)
    m = _silu(v * mW1_ref[...] + mb1_ref[...])
    mod = jnp.dot(m.astype(bf16), mW2_ref[...],
                  preferred_element_type=f32) + mb2_ref[...]
    scale = jax.nn.sigmoid(mod[:, :D])
    shift = mod[:, D:]
    t0 = tf_tab_ref[0:1, :]
    t1 = tf_tab_ref[1:2, :]
    tf_emb = t0 + tf_ref[...] * (t1 - t0)
    scale_ref[...] = scale
    partial_ref[...] = (expr + tf_emb + gene_type_ref[...]) * scale + shift


def _combine_body(gene_ref, scale_ref, partial_ref, validt_ref, ce_ref,
                  ctxW1_ref, ctxb1_ref, ctxW2_ref, ctxb2_ref,
                  pW1_ref, pb1_ref, pW2_ref, pb2_ref,
                  xW1_ref, xb1_ref, xW2_ref, xb2_ref,
                  prefix_type_ref, rms_ref, pos_ref, out_ref):
    f32 = jnp.float32
    tokens = gene_ref[0] * scale_ref[0] + partial_ref[0]
    vrow = validt_ref[0]
    pooled = jnp.dot(vrow, tokens, preferred_element_type=f32) / jnp.maximum(
        jnp.sum(vrow), 1.0)
    ctx = jnp.dot(_silu(jnp.dot(pooled, ctxW1_ref[...],
                                preferred_element_type=f32) + ctxb1_ref[...]),
                  ctxW2_ref[...], preferred_element_type=f32) + ctxb2_ref[...]
    ce = ce_ref[0]
    ptok = jnp.dot(_silu(jnp.dot(ce, pW1_ref[...],
                                 preferred_element_type=f32) + pb1_ref[...]),
                   pW2_ref[...], preferred_element_type=f32) + pb2_ref[...]
    cbias = jnp.dot(_silu(jnp.dot(ce, xW1_ref[...],
                                  preferred_element_type=f32) + xb1_ref[...]),
                    xW2_ref[...], preferred_element_type=f32) + xb2_ref[...]
    prefix_row = ptok + ctx + prefix_type_ref[...] + pos_ref[0:1, :]
    genes = tokens + cbias + pos_ref[1:, :]
    full = jnp.concatenate([prefix_row, genes], axis=0)
    norm = full * lax.rsqrt(
        jnp.mean(full * full, axis=-1, keepdims=True) + 1e-6) * rms_ref[...]
    out_ref[0] = norm


def kernel(expression_values, gene_table, zero_embedding, eW1, eb1, eW2, eb2,
           eW3, eb3, mW1, mb1, mW2, mb2, cond_table, pW1, pb1, pW2, pb2,
           xW1, xb1, xW2, xb2, ctxW1, ctxb1, ctxW2, ctxb2, tf_table,
           pos_table, prefix_type, gene_type, rms_w, input_ids,
           condition_ids, padding_mask, non_tf_mask):
    B = C * G
    gene_idx = input_ids.reshape(-1).astype(jnp.int32)
    cond_idx = condition_ids.reshape(-1).astype(jnp.int32)

    gathered, ce_rows = _sc_gather(gene_table, gene_idx, cond_table, cond_idx)

    ev2 = expression_values.reshape(B, 1)
    tf2 = non_tf_mask.astype(jnp.float32).reshape(B, 1)
    row = lambda b: b.reshape(1, -1)
    bf = lambda w: w.astype(jnp.bfloat16)

    w1 = (eW1, row(eb1), bf(eW2), row(eb2), bf(eW3), row(eb3),
          mW1, row(mb1), bf(mW2), row(mb2),
          tf_table, gene_type.reshape(1, D), row(zero_embedding))
    full_spec = lambda a: pl.BlockSpec(a.shape, lambda i: (0,) * a.ndim)
    scale_f, partial_f = pl.pallas_call(
        _mlp_body,
        grid=(B // _TB,),
        in_specs=[pl.BlockSpec((_TB, 1), lambda i: (i, 0)),
                  pl.BlockSpec((_TB, 1), lambda i: (i, 0))]
        + [full_spec(w) for w in w1],
        out_specs=(pl.BlockSpec((_TB, D), lambda i: (i, 0)),
                   pl.BlockSpec((_TB, D), lambda i: (i, 0))),
        out_shape=(jax.ShapeDtypeStruct((B, D), jnp.float32),
                   jax.ShapeDtypeStruct((B, D), jnp.float32)),
    )(ev2, tf2, *w1)

    gathered = gathered.reshape(C, G, D)
    ce3 = ce_rows.reshape(C, 1, 4 * D)
    scale3 = scale_f.reshape(C, G, D)
    partial3 = partial_f.reshape(C, G, D)

    validt = (~padding_mask).astype(jnp.float32).reshape(C, 1, G)
    pos = pos_table[: G + 1]

    w2 = (ctxW1, row(ctxb1), ctxW2, row(ctxb2),
          pW1, row(pb1), pW2, row(pb2),
          xW1, row(xb1), xW2, row(xb2),
          prefix_type.reshape(1, D), row(rms_w), pos)
    in_specs = [
        pl.BlockSpec((1, G, D), lambda c: (c, 0, 0)),
        pl.BlockSpec((1, G, D), lambda c: (c, 0, 0)),
        pl.BlockSpec((1, G, D), lambda c: (c, 0, 0)),
        pl.BlockSpec((1, 1, G), lambda c: (c, 0, 0)),
        pl.BlockSpec((1, 1, 4 * D), lambda c: (c, 0, 0)),
    ] + [full_spec(w) for w in w2]

    out = pl.pallas_call(
        _combine_body,
        grid=(C,),
        in_specs=in_specs,
        out_specs=pl.BlockSpec((1, G + 1, D), lambda c: (c, 0, 0)),
        out_shape=jax.ShapeDtypeStruct((C, G + 1, D), jnp.float32),
    )(gathered, scale3, partial3, validt, ce3, *w2)
    return out


# speedup vs baseline: 3.5892x; 1.1316x over previous
import functools

import jax
import jax.numpy as jnp
from jax import lax
from jax.experimental import pallas as pl
from jax.experimental.pallas import tpu as pltpu
from jax.experimental.pallas import tpu_sc as plsc

C, G, D = 32, 2048, 256
_NC, _NS = 2, 16
_NW = _NC * _NS
_CH = 128


def _silu(x):
    return x * jax.nn.sigmoid(x)


def _sc_gather(gene_table, gene_idx, cond_table, cond_idx):
    B = gene_idx.shape[0]
    CB = cond_idx.shape[0]
    rows_w = B // _NW
    nch = rows_w // _CH

    mesh = plsc.VectorSubcoreMesh(core_axis_name="c", subcore_axis_name="s")

    @functools.partial(
        pl.kernel,
        out_type=(
            jax.ShapeDtypeStruct((B, D), jnp.float32),
            jax.ShapeDtypeStruct((CB, D), jnp.float32),
        ),
        mesh=mesh,
        scratch_types=[
            pltpu.VMEM((rows_w,), jnp.int32),
            pltpu.VMEM((_CH, D), jnp.float32),
            pltpu.VMEM((_CH, D), jnp.float32),
            pltpu.VMEM((CB,), jnp.int32),
            pltpu.VMEM((CB, D), jnp.float32),
            pltpu.SemaphoreType.DMA,
            pltpu.SemaphoreType.DMA,
            pltpu.SemaphoreType.DMA,
            pltpu.SemaphoreType.DMA,
            pltpu.SemaphoreType.DMA,
        ],
    )
    def gather_k(table_h, idx_h, ctab_h, cidx_h, out_h, cout_h,
                 idx_v, rows_a, rows_b, cidx_v, crows_v,
                 gsem_a, gsem_b, ssem_a, ssem_b, csem):
        wid = lax.axis_index("s") * _NC + lax.axis_index("c")
        base = wid * rows_w
        pltpu.sync_copy(idx_h.at[pl.ds(base, rows_w)], idx_v)

        bufs = (rows_a, rows_b)
        gsems = (gsem_a, gsem_b)
        ssems = (ssem_a, ssem_b)
        gcp = [None, None]
        scp = [None, None]
        for k in range(2):
            gcp[k] = pltpu.async_copy(
                table_h.at[idx_v.at[pl.ds(k * _CH, _CH)]], bufs[k], gsems[k])
        for k in range(nch):
            b = k % 2
            gcp[b].wait()
            scp[b] = pltpu.async_copy(
                bufs[b], out_h.at[pl.ds(base + k * _CH, _CH)], ssems[b])
            if k + 2 < nch:
                scp[b].wait()
                gcp[b] = pltpu.async_copy(
                    table_h.at[idx_v.at[pl.ds((k + 2) * _CH, _CH)]],
                    bufs[b], gsems[b])
        scp[0].wait()
        scp[1].wait()

        @pl.when(wid == 0)
        def _():
            pltpu.sync_copy(cidx_h, cidx_v)
            pltpu.async_copy(ctab_h.at[cidx_v], crows_v, csem).wait()
            pltpu.sync_copy(crows_v, cout_h)

    return gather_k(gene_table, gene_idx, cond_table, cond_idx)


def _cell_body(gene_ref, ev_ref, tf_ref, validt_ref, ce_ref,
               W1cat_ref, b1cat_ref, eW2_ref, eb2_ref, eW3_ref, eb3_ref,
               mW2_ref, mb2_ref,
               tf_tab_ref, zero_ref,
               ctxW1_ref, ctxb1_ref, ctxW2_ref, ctxb2_ref,
               pW1_ref, pb1_ref, pW2_ref, pb2_ref,
               xW1_ref, xb1_ref, xW2_ref, xb2_ref,
               prefix_type_ref, rms_ref, pos_ref, out_ref):
    f32 = jnp.float32
    bf16 = jnp.bfloat16
    v = ev_ref[0]
    s = _silu(v * W1cat_ref[...] + b1cat_ref[...])
    h = s[:, :D]
    m = s[:, D:]
    h = _silu(jnp.dot(h.astype(bf16), eW2_ref[...],
                      preferred_element_type=f32) + eb2_ref[...])
    expr = jnp.dot(h.astype(bf16), eW3_ref[...],
                   preferred_element_type=f32) + eb3_ref[...]
    expr = jnp.where(v == 0.0, zero_ref[...], expr)
    mod = jnp.dot(m.astype(bf16), mW2_ref[...],
                  preferred_element_type=f32) + mb2_ref[...]
    scale = jax.nn.sigmoid(mod[:, :D])
    shift = mod[:, D:]
    t0 = tf_tab_ref[0:1, :]
    t1 = tf_tab_ref[1:2, :]
    tf_emb = t0 + tf_ref[0] * (t1 - t0)
    tokens = (gene_ref[0] + expr + tf_emb) * scale + shift
    vrow = validt_ref[0]
    pooled = jnp.dot(vrow, tokens, preferred_element_type=f32) / jnp.maximum(
        jnp.sum(vrow), 1.0)
    ctx = jnp.dot(_silu(jnp.dot(pooled, ctxW1_ref[...],
                                preferred_element_type=f32) + ctxb1_ref[...]),
                  ctxW2_ref[...], preferred_element_type=f32) + ctxb2_ref[...]
    ce = ce_ref[0]
    ptok = jnp.dot(_silu(jnp.dot(ce, pW1_ref[...],
                                 preferred_element_type=f32) + pb1_ref[...]),
                   pW2_ref[...], preferred_element_type=f32) + pb2_ref[...]
    cbias = jnp.dot(_silu(jnp.dot(ce, xW1_ref[...],
                                  preferred_element_type=f32) + xb1_ref[...]),
                    xW2_ref[...], preferred_element_type=f32) + xb2_ref[...]
    prefix_row = ptok + ctx + prefix_type_ref[...] + pos_ref[0:1, :]
    genes = tokens + cbias + pos_ref[1:, :]
    full = jnp.concatenate([prefix_row, genes], axis=0)
    norm = full * lax.rsqrt(
        jnp.mean(full * full, axis=-1, keepdims=True) + 1e-6) * rms_ref[...]
    out_ref[0] = norm


def kernel(expression_values, gene_table, zero_embedding, eW1, eb1, eW2, eb2,
           eW3, eb3, mW1, mb1, mW2, mb2, cond_table, pW1, pb1, pW2, pb2,
           xW1, xb1, xW2, xb2, ctxW1, ctxb1, ctxW2, ctxb2, tf_table,
           pos_table, prefix_type, gene_type, rms_w, input_ids,
           condition_ids, padding_mask, non_tf_mask):
    gene_idx = input_ids.reshape(-1).astype(jnp.int32)
    cond_idx = condition_ids.reshape(-1).astype(jnp.int32)
    gathered, ce_rows = _sc_gather(gene_table, gene_idx, cond_table, cond_idx)
    gathered = gathered.reshape(C, G, D)
    ce3 = ce_rows.reshape(C, 1, 4 * D)

    ev3 = expression_values.reshape(C, G, 1)
    tf3 = non_tf_mask.astype(jnp.float32).reshape(C, G, 1)
    validt = (~padding_mask).astype(jnp.float32).reshape(C, 1, G)
    pos = pos_table[: G + 1]

    row = lambda b: b.reshape(1, -1)
    bf = lambda w: w.astype(jnp.bfloat16)
    W1cat = jnp.concatenate([eW1, mW1], axis=1)
    b1cat = jnp.concatenate([eb1, mb1]).reshape(1, -1)
    tf_tab_adj = tf_table + gene_type.reshape(1, D)
    weights = (W1cat, b1cat, bf(eW2), row(eb2), bf(eW3), row(eb3),
               bf(mW2), row(mb2),
               tf_tab_adj, row(zero_embedding),
               ctxW1, row(ctxb1), ctxW2, row(ctxb2),
               pW1, row(pb1), pW2, row(pb2),
               xW1, row(xb1), xW2, row(xb2),
               prefix_type.reshape(1, D), row(rms_w), pos)

    full_spec = lambda a: pl.BlockSpec(a.shape, lambda c: (0,) * a.ndim)
    in_specs = [
        pl.BlockSpec((1, G, D), lambda c: (c, 0, 0)),
        pl.BlockSpec((1, G, 1), lambda c: (c, 0, 0)),
        pl.BlockSpec((1, G, 1), lambda c: (c, 0, 0)),
        pl.BlockSpec((1, 1, G), lambda c: (c, 0, 0)),
        pl.BlockSpec((1, 1, 4 * D), lambda c: (c, 0, 0)),
    ] + [full_spec(w) for w in weights]

    out = pl.pallas_call(
        _cell_body,
        grid=(C,),
        in_specs=in_specs,
        out_specs=pl.BlockSpec((1, G + 1, D), lambda c: (c, 0, 0)),
        out_shape=jax.ShapeDtypeStruct((C, G + 1, D), jnp.float32),
    )(gathered, ev3, tf3, validt, ce3, *weights)
    return out
